# baseline (speedup 1.0000x reference)
import functools

import jax
import jax.numpy as jnp
from jax import lax
from jax.experimental import pallas as pl
from jax.experimental.pallas import tpu as pltpu
from jax.experimental.pallas import tpu_sc as plsc

N_NODES = 10000
N_EDGES = 320000
F = 128
L = 16

NC = 2
NS = 16
NPT = 632
NPAD = NS * NPT
E_PER_SC = N_EDGES // NC
CE = 4000
NCHUNK = E_PER_SC // CE
MB = 4224
GB = 128


_ROWS_BLK = 2000


def _mm_kernel(x_ref, tw_ref, pw_ref, tb_ref, pb_ref, a_ref, b_ref):
    x = x_ref[...]
    tw = tw_ref[...]
    dn = (((1,), (1,)), ((), ()))
    a_ref[...] = (
        lax.dot_general(x, tw, dn, preferred_element_type=jnp.float32)
        + tb_ref[...] + pb_ref[...]
    )
    w = pw_ref[...] - tw
    b0 = lax.dot_general(x, w[:64], dn, preferred_element_type=jnp.float32)
    b1 = lax.dot_general(x, w[64:], dn, preferred_element_type=jnp.float32)
    u0 = lax.bitcast_convert_type(
        b0.astype(jnp.bfloat16), jnp.uint16
    ).astype(jnp.uint32)
    u1 = lax.bitcast_convert_type(
        b1.astype(jnp.bfloat16), jnp.uint16
    ).astype(jnp.uint32)
    b_ref[...] = u0 | (u1 << jnp.uint32(16))


def _node_matmuls(x, theta_w, phi_w, theta_b, phi_b):
    n = x.shape[0]
    grid = n // _ROWS_BLK
    return pl.pallas_call(
        _mm_kernel,
        grid=(grid,),
        in_specs=[
            pl.BlockSpec((_ROWS_BLK, F), lambda i: (i, 0)),
            pl.BlockSpec((F, F), lambda i: (0, 0)),
            pl.BlockSpec((F, F), lambda i: (0, 0)),
            pl.BlockSpec((1, F), lambda i: (0, 0)),
            pl.BlockSpec((1, F), lambda i: (0, 0)),
        ],
        out_specs=[
            pl.BlockSpec((_ROWS_BLK, F), lambda i: (i, 0)),
            pl.BlockSpec((_ROWS_BLK, F // 2), lambda i: (i, 0)),
        ],
        out_shape=[
            jax.ShapeDtypeStruct((n, F), jnp.float32),
            jax.ShapeDtypeStruct((n, F // 2), jnp.uint32),
        ],
    )(x, theta_w, phi_w, theta_b, phi_b)


def _sc_body(b_hbm, src_hbm, dst_hbm, out_hbm, c_loc,
             src_ch0, dst_ch0, src_ch1, dst_ch1, src_m, dst_m,
             rows0, rows1, sem_s0, sem_d0, sem_s1, sem_d1, sem_g0, sem_g1):
    c = lax.axis_index("c")
    s = lax.axis_index("s")
    lo = s * NPT

    src_ch = (src_ch0, src_ch1)
    dst_ch = (dst_ch0, dst_ch1)
    sem_s = (sem_s0, sem_s1)
    sem_d = (sem_d0, sem_d1)
    rows = (rows0, rows1)
    sem_g = (sem_g0, sem_g1)

    neg_inf = jnp.full((L,), -jnp.inf, dtype=jnp.float32)

    def init_row(r, carry):
        for j in range(F // L):
            c_loc[r, pl.ds(j * L, L)] = neg_inf
        return carry

    lax.fori_loop(0, NPT + 1, init_row, 0)

    zero_v = jnp.zeros((L,), dtype=jnp.int32)

    def init_idx(g, carry):
        src_m[pl.ds(g * L, L)] = zero_v
        return carry

    lax.fori_loop(0, MB // L, init_idx, 0)

    lane = lax.iota(jnp.int32, L)
    sentv = jnp.full((L,), NPT, dtype=jnp.int32)
    ebase = c * E_PER_SC

    def fire_chunk(k, buf):
        off = ebase + k * CE
        pltpu.async_copy(src_hbm.at[pl.ds(off, CE)], src_ch[buf], sem_s[buf])
        pltpu.async_copy(dst_hbm.at[pl.ds(off, CE)], dst_ch[buf], sem_d[buf])

    def wait_chunk(k, buf):
        off = ebase + k * CE
        pltpu.make_async_copy(
            src_hbm.at[pl.ds(off, CE)], src_ch[buf], sem_s[buf]
        ).wait()
        pltpu.make_async_copy(
            dst_hbm.at[pl.ds(off, CE)], dst_ch[buf], sem_d[buf]
        ).wait()

    def fire_gather(b, rbuf):
        pltpu.async_copy(
            b_hbm.at[src_m.at[pl.ds(b * GB, GB)]], rows[rbuf], sem_g[rbuf]
        )

    def wait_gather(b, rbuf):
        pltpu.make_async_copy(
            b_hbm.at[src_m.at[pl.ds(b * GB, GB)]], rows[rbuf], sem_g[rbuf]
        ).wait()

    def process_batch(b, rbuf):
        rbuf_ref = rows[rbuf]

        def group_body(g, carry2):
            e0 = b * GB + g * L
            dv = dst_m[pl.ds(e0, L)]
            for i in range(L):
                bc = jnp.take_along_axis(
                    dv,
                    jnp.full((L,), i, dtype=jnp.int32),
                    axis=0,
                    mode="promise_in_bounds",
                )
                for j in range(F // (2 * L)):
                    u = rbuf_ref[g * L + i, pl.ds(j * L, L)]
                    lo_f = plsc.bitcast(u << jnp.uint32(16), jnp.float32)
                    hi_f = plsc.bitcast(
                        u & jnp.uint32(0xFFFF0000), jnp.float32
                    )
                    col_lo = lane + (j * L)
                    col_hi = col_lo + (F // 2)
                    cur = plsc.load_gather(c_loc, [bc, col_lo])
                    plsc.store_scatter(
                        c_loc, [bc, col_lo], jnp.maximum(cur, lo_f)
                    )
                    cur2 = plsc.load_gather(c_loc, [bc, col_hi])
                    plsc.store_scatter(
                        c_loc, [bc, col_hi], jnp.maximum(cur2, hi_f)
                    )
            return carry2

        lax.fori_loop(0, GB // L, group_body, 0)

    def do_chunk(k, buf):
        wait_chunk(k, buf)

        def scan_pair(gp, cnt_v):
            for h in range(2):
                e0 = gp * (2 * L) + h * L
                dv = dst_ch[buf][pl.ds(e0, L)]
                sv = src_ch[buf][pl.ds(e0, L)]
                dl = dv - lo
                msk = dl.astype(jnp.uint32) < jnp.uint32(NPT)
                inc = plsc.cumsum(msk.astype(jnp.int32))
                pos = cnt_v + inc - 1
                plsc.store_scatter(src_m, [pos], sv, mask=msk)
                plsc.store_scatter(dst_m, [pos], dl, mask=msk)
                cnt_v = cnt_v + plsc.all_reduce_population_count(msk)
            return cnt_v

        cnt_v = lax.fori_loop(
            0, CE // (2 * L), scan_pair, jnp.zeros((L,), jnp.int32)
        )
        cnt = jnp.max(cnt_v)

        for t in range(GB // L):
            plsc.store_scatter(dst_m, [cnt + lane + t * L], sentv)

        nb = (cnt + GB - 1) // GB

        @pl.when(nb > 0)
        def _():
            fire_gather(0, 0)

            def pair_body(p, carry):
                b0 = 2 * p

                @pl.when(b0 + 1 < nb)
                def _():
                    fire_gather(b0 + 1, 1)

                wait_gather(b0, 0)
                process_batch(b0, 0)

                @pl.when(b0 + 1 < nb)
                def _():
                    @pl.when(b0 + 2 < nb)
                    def _():
                        fire_gather(b0 + 2, 0)

                    wait_gather(b0 + 1, 1)
                    process_batch(b0 + 1, 1)

                return carry

            lax.fori_loop(0, (nb + 1) // 2, pair_body, 0)

    fire_chunk(0, 0)

    def pair_chunks(p, carry):
        a = 2 * p
        fire_chunk(a + 1, 1)
        do_chunk(a, 0)

        @pl.when(a + 2 < NCHUNK)
        def _():
            fire_chunk(a + 2, 0)

        do_chunk(a + 1, 1)
        return carry

    lax.fori_loop(0, NCHUNK // 2, pair_chunks, 0)

    pltpu.sync_copy(
        c_loc.at[pl.ds(0, NPT)], out_hbm.at[c, pl.ds(lo, NPT)]
    )


_sc_scatter_max = functools.partial(
    pl.kernel,
    out_type=jax.ShapeDtypeStruct((NC, NPAD, F), jnp.float32),
    mesh=plsc.VectorSubcoreMesh(core_axis_name="c", subcore_axis_name="s"),
    compiler_params=pltpu.CompilerParams(
        needs_layout_passes=False, use_tc_tiling_on_sc=False
    ),
    scratch_types=[
        pltpu.VMEM((NPT + 1, F), jnp.float32),
        pltpu.VMEM((CE,), jnp.int32),
        pltpu.VMEM((CE,), jnp.int32),
        pltpu.VMEM((CE,), jnp.int32),
        pltpu.VMEM((CE,), jnp.int32),
        pltpu.VMEM((MB,), jnp.int32),
        pltpu.VMEM((MB,), jnp.int32),
        pltpu.VMEM((GB, F // 2), jnp.uint32),
        pltpu.VMEM((GB, F // 2), jnp.uint32),
        pltpu.SemaphoreType.DMA,
        pltpu.SemaphoreType.DMA,
        pltpu.SemaphoreType.DMA,
        pltpu.SemaphoreType.DMA,
        pltpu.SemaphoreType.DMA,
        pltpu.SemaphoreType.DMA,
    ],
)(_sc_body)


def _combine_kernel(a_ref, c_ref, o_ref):
    cm = jnp.maximum(c_ref[0], c_ref[1])
    o_ref[...] = jnp.where(jnp.isfinite(cm), a_ref[...] + cm, 0.0)


def _combine(a, c_part):
    n = a.shape[0]
    grid = n // _ROWS_BLK
    return pl.pallas_call(
        _combine_kernel,
        grid=(grid,),
        in_specs=[
            pl.BlockSpec((_ROWS_BLK, F), lambda i: (i, 0)),
            pl.BlockSpec((NC, _ROWS_BLK, F), lambda i: (0, i, 0)),
        ],
        out_specs=pl.BlockSpec((_ROWS_BLK, F), lambda i: (i, 0)),
        out_shape=jax.ShapeDtypeStruct((n, F), jnp.float32),
    )(a, c_part)


@jax.jit
def kernel(h, edge_index, theta_w, theta_b, phi_w, phi_b):
    n_samples, n_points, n_dims = h.shape
    x = h.reshape(-1, n_dims)
    a, b = _node_matmuls(
        x,
        theta_w,
        phi_w,
        theta_b.reshape(1, F),
        phi_b.reshape(1, F),
    )
    c_part = _sc_scatter_max(b, edge_index[0], edge_index[1])
    out = _combine(a, c_part)
    return out.reshape(n_samples, n_points, F)


# speedup vs baseline: 2.4377x; 2.4377x over previous
import functools

import jax
import jax.numpy as jnp
from jax import lax
from jax.experimental import pallas as pl
from jax.experimental.pallas import tpu as pltpu
from jax.experimental.pallas import tpu_sc as plsc

N_NODES = 10000
N_EDGES = 320000
F = 128
L = 16

NC = 2
NS = 16
NPT = 632
NPAD = NS * NPT
E_PER_SC = N_EDGES // NC
CE = 4000
NCHUNK = E_PER_SC // CE
MB = 4160
GB = 128
NEG_INF_PAIR = -8323200


_ROWS_BLK = 2000


def _mm_kernel(x_ref, tw_ref, pw_ref, tb_ref, pb_ref, a_ref, b_ref):
    x = x_ref[...]
    tw = tw_ref[...]
    dn = (((1,), (1,)), ((), ()))
    a_ref[...] = (
        lax.dot_general(x, tw, dn, preferred_element_type=jnp.float32)
        + tb_ref[...] + pb_ref[...]
    )
    w = pw_ref[...] - tw
    b0 = lax.dot_general(x, w[:64], dn, preferred_element_type=jnp.float32)
    b1 = lax.dot_general(x, w[64:], dn, preferred_element_type=jnp.float32)
    u0 = lax.bitcast_convert_type(
        b0.astype(jnp.bfloat16), jnp.uint16
    ).astype(jnp.int32)
    u1 = lax.bitcast_convert_type(
        b1.astype(jnp.bfloat16), jnp.uint16
    ).astype(jnp.int32)
    b_ref[...] = u0 | (u1 << jnp.int32(16))


def _node_matmuls(x, theta_w, phi_w, theta_b, phi_b):
    n = x.shape[0]
    grid = n // _ROWS_BLK
    return pl.pallas_call(
        _mm_kernel,
        grid=(grid,),
        in_specs=[
            pl.BlockSpec((_ROWS_BLK, F), lambda i: (i, 0)),
            pl.BlockSpec((F, F), lambda i: (0, 0)),
            pl.BlockSpec((F, F), lambda i: (0, 0)),
            pl.BlockSpec((1, F), lambda i: (0, 0)),
            pl.BlockSpec((1, F), lambda i: (0, 0)),
        ],
        out_specs=[
            pl.BlockSpec((_ROWS_BLK, F), lambda i: (i, 0)),
            pl.BlockSpec((_ROWS_BLK, F // 2), lambda i: (i, 0)),
        ],
        out_shape=[
            jax.ShapeDtypeStruct((n, F), jnp.float32),
            jax.ShapeDtypeStruct((n, F // 2), jnp.int32),
        ],
    )(x, theta_w, phi_w, theta_b, phi_b)


def _sc_body(b_hbm, src_hbm, dst_hbm, out_hbm, c_loc, shared_b,
             src_ch0, dst_ch0, src_ch1, dst_ch1, src_m, dst_m,
             rows0, rows1, sem_s0, sem_d0, sem_s1, sem_d1, sem_g0, sem_g1):
    c = lax.axis_index("c")
    s = lax.axis_index("s")
    lo = s * NPT

    rpt = N_NODES // NS
    pltpu.sync_copy(
        b_hbm.at[pl.ds(s * rpt, rpt)], shared_b.at[pl.ds(s * rpt, rpt)]
    )
    plsc.subcore_barrier()

    src_ch = (src_ch0, src_ch1)
    dst_ch = (dst_ch0, dst_ch1)
    sem_s = (sem_s0, sem_s1)
    sem_d = (sem_d0, sem_d1)
    rows = (rows0, rows1)
    sem_g = (sem_g0, sem_g1)

    neg_inf = jnp.full((L,), NEG_INF_PAIR, dtype=jnp.int32)

    def init_row(r, carry):
        for j in range(F // (2 * L)):
            c_loc[r, pl.ds(j * L, L)] = neg_inf
        return carry

    lax.fori_loop(0, NPT + 1, init_row, 0)

    zero_v = jnp.zeros((L,), dtype=jnp.int32)

    def init_idx(g, carry):
        src_m[pl.ds(g * L, L)] = zero_v
        return carry

    lax.fori_loop(0, MB // L, init_idx, 0)

    lane = lax.iota(jnp.int32, L)
    sentv = jnp.full((L,), NPT, dtype=jnp.int32)
    ebase = c * E_PER_SC

    def fire_chunk(k, buf):
        off = ebase + k * CE
        pltpu.async_copy(src_hbm.at[pl.ds(off, CE)], src_ch[buf], sem_s[buf])
        pltpu.async_copy(dst_hbm.at[pl.ds(off, CE)], dst_ch[buf], sem_d[buf])

    def wait_chunk(k, buf):
        off = ebase + k * CE
        pltpu.make_async_copy(
            src_hbm.at[pl.ds(off, CE)], src_ch[buf], sem_s[buf]
        ).wait()
        pltpu.make_async_copy(
            dst_hbm.at[pl.ds(off, CE)], dst_ch[buf], sem_d[buf]
        ).wait()

    def fire_gather(b, rbuf):
        pltpu.async_copy(
            shared_b.at[src_m.at[pl.ds(b * GB, GB)]], rows[rbuf], sem_g[rbuf]
        )

    def wait_gather(b, rbuf):
        pltpu.make_async_copy(
            shared_b.at[src_m.at[pl.ds(b * GB, GB)]], rows[rbuf], sem_g[rbuf]
        ).wait()

    def process_batch(b, rbuf):
        rbuf_ref = rows[rbuf]

        def group_body(g, carry2):
            e0 = b * GB + g * L
            dv = dst_m[pl.ds(e0, L)]
            for i in range(L):
                bc = jnp.take_along_axis(
                    dv,
                    jnp.full((L,), i, dtype=jnp.int32),
                    axis=0,
                    mode="promise_in_bounds",
                )
                for j in range(F // (2 * L)):
                    u = rbuf_ref[g * L + i, pl.ds(j * L, L)]
                    rv = plsc.bitcast(u, jnp.bfloat16)
                    colv = lane + (j * L)
                    cur_u = plsc.load_gather(c_loc, [bc, colv])
                    cur = plsc.bitcast(cur_u, jnp.bfloat16)
                    mx = jnp.maximum(cur, rv)
                    plsc.store_scatter(
                        c_loc, [bc, colv], plsc.bitcast(mx, jnp.int32)
                    )
            return carry2

        lax.fori_loop(0, GB // L, group_body, 0)

    def do_chunk(k, buf):
        wait_chunk(k, buf)

        def scan_pair(gp, cnt_v):
            for h in range(2):
                e0 = gp * (2 * L) + h * L
                dv = dst_ch[buf][pl.ds(e0, L)]
                sv = src_ch[buf][pl.ds(e0, L)]
                dl = dv - lo
                msk = dl.astype(jnp.uint32) < jnp.uint32(NPT)
                inc = plsc.cumsum(msk.astype(jnp.int32))
                pos = cnt_v + inc - 1
                plsc.store_scatter(src_m, [pos], sv, mask=msk)
                plsc.store_scatter(dst_m, [pos], dl, mask=msk)
                cnt_v = cnt_v + plsc.all_reduce_population_count(msk)
            return cnt_v

        cnt_v = lax.fori_loop(
            0, CE // (2 * L), scan_pair, jnp.zeros((L,), jnp.int32)
        )
        cnt = jnp.max(cnt_v)

        for t in range(GB // L):
            plsc.store_scatter(dst_m, [cnt + lane + t * L], sentv)

        nb = (cnt + GB - 1) // GB

        @pl.when(nb > 0)
        def _():
            fire_gather(0, 0)

            def pair_body(p, carry):
                b0 = 2 * p

                @pl.when(b0 + 1 < nb)
                def _():
                    fire_gather(b0 + 1, 1)

                wait_gather(b0, 0)
                process_batch(b0, 0)

                @pl.when(b0 + 1 < nb)
                def _():
                    @pl.when(b0 + 2 < nb)
                    def _():
                        fire_gather(b0 + 2, 0)

                    wait_gather(b0 + 1, 1)
                    process_batch(b0 + 1, 1)

                return carry

            lax.fori_loop(0, (nb + 1) // 2, pair_body, 0)

    fire_chunk(0, 0)

    def pair_chunks(p, carry):
        a = 2 * p
        fire_chunk(a + 1, 1)
        do_chunk(a, 0)

        @pl.when(a + 2 < NCHUNK)
        def _():
            fire_chunk(a + 2, 0)

        do_chunk(a + 1, 1)
        return carry

    lax.fori_loop(0, NCHUNK // 2, pair_chunks, 0)

    pltpu.sync_copy(
        c_loc.at[pl.ds(0, NPT)], out_hbm.at[c, pl.ds(lo, NPT)]
    )


_sc_scatter_max = functools.partial(
    pl.kernel,
    out_type=jax.ShapeDtypeStruct((NC, NPAD, F // 2), jnp.int32),
    mesh=plsc.VectorSubcoreMesh(core_axis_name="c", subcore_axis_name="s"),
    compiler_params=pltpu.CompilerParams(
        needs_layout_passes=False, use_tc_tiling_on_sc=False
    ),
    scratch_types=[
        pltpu.VMEM((NPT + 1, F // 2), jnp.int32),
        pltpu.VMEM_SHARED((N_NODES, F // 2), jnp.int32),
        pltpu.VMEM((CE,), jnp.int32),
        pltpu.VMEM((CE,), jnp.int32),
        pltpu.VMEM((CE,), jnp.int32),
        pltpu.VMEM((CE,), jnp.int32),
        pltpu.VMEM((MB,), jnp.int32),
        pltpu.VMEM((MB,), jnp.int32),
        pltpu.VMEM((GB, F // 2), jnp.int32),
        pltpu.VMEM((GB, F // 2), jnp.int32),
        pltpu.SemaphoreType.DMA,
        pltpu.SemaphoreType.DMA,
        pltpu.SemaphoreType.DMA,
        pltpu.SemaphoreType.DMA,
        pltpu.SemaphoreType.DMA,
        pltpu.SemaphoreType.DMA,
    ],
)(_sc_body)


def _combine_kernel(a_ref, c_ref, o_ref):
    cm = jnp.maximum(c_ref[0], c_ref[1]).astype(jnp.float32)
    o_ref[...] = jnp.where(jnp.isfinite(cm), a_ref[...] + cm, 0.0)


def _combine(a, c_nat):
    n = a.shape[0]
    grid = n // _ROWS_BLK
    return pl.pallas_call(
        _combine_kernel,
        grid=(grid,),
        in_specs=[
            pl.BlockSpec((_ROWS_BLK, F), lambda i: (i, 0)),
            pl.BlockSpec((NC, _ROWS_BLK, F), lambda i: (0, i, 0)),
        ],
        out_specs=pl.BlockSpec((_ROWS_BLK, F), lambda i: (i, 0)),
        out_shape=jax.ShapeDtypeStruct((n, F), jnp.float32),
    )(a, c_nat)


@jax.jit
def kernel(h, edge_index, theta_w, theta_b, phi_w, phi_b):
    n_samples, n_points, n_dims = h.shape
    x = h.reshape(-1, n_dims)
    a, b = _node_matmuls(
        x,
        theta_w,
        phi_w,
        theta_b.reshape(1, F),
        phi_b.reshape(1, F),
    )
    c_part = _sc_scatter_max(b, edge_index[0], edge_index[1])
    c_bf = lax.bitcast_convert_type(c_part, jnp.bfloat16)
    c_nat = jnp.concatenate([c_bf[..., 0], c_bf[..., 1]], axis=-1)
    out = _combine(a, c_nat)
    return out.reshape(n_samples, n_points, F)


# speedup vs baseline: 2.4515x; 1.0057x over previous
import functools

import jax
import jax.numpy as jnp
from jax import lax
from jax.experimental import pallas as pl
from jax.experimental.pallas import tpu as pltpu
from jax.experimental.pallas import tpu_sc as plsc

N_NODES = 10000
N_EDGES = 320000
F = 128
L = 16

NC = 2
NS = 16
NPT = 632
NPAD = NS * NPT
E_PER_SC = N_EDGES // NC
CE = 4000
NCHUNK = E_PER_SC // CE
MB = 4160
GB = 128
NEG_INF_PAIR = -8323200


_ROWS_BLK = 2000


def _mm_kernel(x_ref, tw_ref, pw_ref, tb_ref, pb_ref, a_ref, b_ref):
    x = x_ref[...]
    tw = tw_ref[...]
    dn = (((1,), (1,)), ((), ()))
    a_ref[...] = (
        lax.dot_general(x, tw, dn, preferred_element_type=jnp.float32)
        + tb_ref[...] + pb_ref[...]
    )
    w = pw_ref[...] - tw
    b0 = lax.dot_general(x, w[:64], dn, preferred_element_type=jnp.float32)
    b1 = lax.dot_general(x, w[64:], dn, preferred_element_type=jnp.float32)
    u0 = lax.bitcast_convert_type(
        b0.astype(jnp.bfloat16), jnp.uint16
    ).astype(jnp.int32)
    u1 = lax.bitcast_convert_type(
        b1.astype(jnp.bfloat16), jnp.uint16
    ).astype(jnp.int32)
    b_ref[...] = u0 | (u1 << jnp.int32(16))


def _node_matmuls(x, theta_w, phi_w, theta_b, phi_b):
    n = x.shape[0]
    grid = n // _ROWS_BLK
    return pl.pallas_call(
        _mm_kernel,
        grid=(grid,),
        in_specs=[
            pl.BlockSpec((_ROWS_BLK, F), lambda i: (i, 0)),
            pl.BlockSpec((F, F), lambda i: (0, 0)),
            pl.BlockSpec((F, F), lambda i: (0, 0)),
            pl.BlockSpec((1, F), lambda i: (0, 0)),
            pl.BlockSpec((1, F), lambda i: (0, 0)),
        ],
        out_specs=[
            pl.BlockSpec((_ROWS_BLK, F), lambda i: (i, 0)),
            pl.BlockSpec((_ROWS_BLK, F // 2), lambda i: (i, 0)),
        ],
        out_shape=[
            jax.ShapeDtypeStruct((n, F), jnp.float32),
            jax.ShapeDtypeStruct((n, F // 2), jnp.int32),
        ],
    )(x, theta_w, phi_w, theta_b, phi_b)


_EROWS = N_EDGES // F


def _pack_kernel(e_ref, p_ref):
    e = e_ref[...]
    p_ref[...] = (e[1] << jnp.int32(16)) | e[0]


def _pack_edges(edge_index):
    e3 = edge_index.reshape(2, _EROWS, F)
    packed = pl.pallas_call(
        _pack_kernel,
        out_shape=jax.ShapeDtypeStruct((_EROWS, F), jnp.int32),
    )(e3)
    return packed.reshape(N_EDGES)


def _sc_body(b_hbm, edge_hbm, out_hbm, c_loc, shared_b,
             ech0, ech1, src_m, dst_m,
             rows0, rows1, sem_e0, sem_e1, sem_g0, sem_g1):
    c = lax.axis_index("c")
    s = lax.axis_index("s")
    lo = s * NPT

    rpt = N_NODES // NS
    pltpu.sync_copy(
        b_hbm.at[pl.ds(s * rpt, rpt)], shared_b.at[pl.ds(s * rpt, rpt)]
    )
    plsc.subcore_barrier()

    ech = (ech0, ech1)
    sem_e = (sem_e0, sem_e1)
    rows = (rows0, rows1)
    sem_g = (sem_g0, sem_g1)

    neg_inf = jnp.full((L,), NEG_INF_PAIR, dtype=jnp.int32)

    def init_row(r, carry):
        for j in range(F // (2 * L)):
            c_loc[r, pl.ds(j * L, L)] = neg_inf
        return carry

    lax.fori_loop(0, NPT + 1, init_row, 0)

    zero_v = jnp.zeros((L,), dtype=jnp.int32)

    def init_idx(g, carry):
        src_m[pl.ds(g * L, L)] = zero_v
        return carry

    lax.fori_loop(0, MB // L, init_idx, 0)

    lane = lax.iota(jnp.int32, L)
    sentv = jnp.full((L,), NPT, dtype=jnp.int32)
    ebase = c * E_PER_SC

    def fire_chunk(k, buf):
        off = ebase + k * CE
        pltpu.async_copy(edge_hbm.at[pl.ds(off, CE)], ech[buf], sem_e[buf])

    def wait_chunk(k, buf):
        off = ebase + k * CE
        pltpu.make_async_copy(
            edge_hbm.at[pl.ds(off, CE)], ech[buf], sem_e[buf]
        ).wait()

    def fire_gather(b, rbuf):
        pltpu.async_copy(
            shared_b.at[src_m.at[pl.ds(b * GB, GB)]], rows[rbuf], sem_g[rbuf]
        )

    def wait_gather(b, rbuf):
        pltpu.make_async_copy(
            shared_b.at[src_m.at[pl.ds(b * GB, GB)]], rows[rbuf], sem_g[rbuf]
        ).wait()

    def process_batch(b, rbuf):
        rbuf_ref = rows[rbuf]

        def group_body(g, carry2):
            e0 = b * GB + g * L
            dv = dst_m[pl.ds(e0, L)]
            for i in range(L):
                bc = jnp.take_along_axis(
                    dv,
                    jnp.full((L,), i, dtype=jnp.int32),
                    axis=0,
                    mode="promise_in_bounds",
                )
                for j in range(F // (2 * L)):
                    u = rbuf_ref[g * L + i, pl.ds(j * L, L)]
                    rv = plsc.bitcast(u, jnp.bfloat16)
                    colv = lane + (j * L)
                    cur_u = plsc.load_gather(c_loc, [bc, colv])
                    cur = plsc.bitcast(cur_u, jnp.bfloat16)
                    mx = jnp.maximum(cur, rv)
                    plsc.store_scatter(
                        c_loc, [bc, colv], plsc.bitcast(mx, jnp.int32)
                    )
            return carry2

        lax.fori_loop(0, GB // L, group_body, 0)

    def do_chunk(k, buf):
        wait_chunk(k, buf)

        def scan_pair(gp, cnt_v):
            for h in range(2):
                e0 = gp * (2 * L) + h * L
                u = ech[buf][pl.ds(e0, L)]
                dv = lax.shift_right_logical(u, jnp.int32(16))
                sv = u & jnp.int32(0xFFFF)
                dl = dv - lo
                msk = dl.astype(jnp.uint32) < jnp.uint32(NPT)
                inc = plsc.cumsum(msk.astype(jnp.int32))
                pos = cnt_v + inc - 1
                plsc.store_scatter(src_m, [pos], sv, mask=msk)
                plsc.store_scatter(dst_m, [pos], dl, mask=msk)
                cnt_v = cnt_v + plsc.all_reduce_population_count(msk)
            return cnt_v

        cnt_v = lax.fori_loop(
            0, CE // (2 * L), scan_pair, jnp.zeros((L,), jnp.int32)
        )
        cnt = jnp.max(cnt_v)

        for t in range(GB // L):
            plsc.store_scatter(dst_m, [cnt + lane + t * L], sentv)

        nb = (cnt + GB - 1) // GB

        @pl.when(nb > 0)
        def _():
            fire_gather(0, 0)

            def pair_body(p, carry):
                b0 = 2 * p

                @pl.when(b0 + 1 < nb)
                def _():
                    fire_gather(b0 + 1, 1)

                wait_gather(b0, 0)
                process_batch(b0, 0)

                @pl.when(b0 + 1 < nb)
                def _():
                    @pl.when(b0 + 2 < nb)
                    def _():
                        fire_gather(b0 + 2, 0)

                    wait_gather(b0 + 1, 1)
                    process_batch(b0 + 1, 1)

                return carry

            lax.fori_loop(0, (nb + 1) // 2, pair_body, 0)

    fire_chunk(0, 0)

    def pair_chunks(p, carry):
        a = 2 * p
        fire_chunk(a + 1, 1)
        do_chunk(a, 0)

        @pl.when(a + 2 < NCHUNK)
        def _():
            fire_chunk(a + 2, 0)

        do_chunk(a + 1, 1)
        return carry

    lax.fori_loop(0, NCHUNK // 2, pair_chunks, 0)

    pltpu.sync_copy(
        c_loc.at[pl.ds(0, NPT)], out_hbm.at[c, pl.ds(lo, NPT)]
    )


_sc_scatter_max = functools.partial(
    pl.kernel,
    out_type=jax.ShapeDtypeStruct((NC, NPAD, F // 2), jnp.int32),
    mesh=plsc.VectorSubcoreMesh(core_axis_name="c", subcore_axis_name="s"),
    compiler_params=pltpu.CompilerParams(
        needs_layout_passes=False, use_tc_tiling_on_sc=False
    ),
    scratch_types=[
        pltpu.VMEM((NPT + 1, F // 2), jnp.int32),
        pltpu.VMEM_SHARED((N_NODES, F // 2), jnp.int32),
        pltpu.VMEM((CE,), jnp.int32),
        pltpu.VMEM((CE,), jnp.int32),
        pltpu.VMEM((MB,), jnp.int32),
        pltpu.VMEM((MB,), jnp.int32),
        pltpu.VMEM((GB, F // 2), jnp.int32),
        pltpu.VMEM((GB, F // 2), jnp.int32),
        pltpu.SemaphoreType.DMA,
        pltpu.SemaphoreType.DMA,
        pltpu.SemaphoreType.DMA,
        pltpu.SemaphoreType.DMA,
    ],
)(_sc_body)


def _combine_kernel(a_ref, c_ref, o_ref):
    cm = jnp.maximum(c_ref[0], c_ref[1]).astype(jnp.float32)
    o_ref[...] = jnp.where(jnp.isfinite(cm), a_ref[...] + cm, 0.0)


def _combine(a, c_nat):
    n = a.shape[0]
    grid = n // _ROWS_BLK
    return pl.pallas_call(
        _combine_kernel,
        grid=(grid,),
        in_specs=[
            pl.BlockSpec((_ROWS_BLK, F), lambda i: (i, 0)),
            pl.BlockSpec((NC, _ROWS_BLK, F), lambda i: (0, i, 0)),
        ],
        out_specs=pl.BlockSpec((_ROWS_BLK, F), lambda i: (i, 0)),
        out_shape=jax.ShapeDtypeStruct((n, F), jnp.float32),
    )(a, c_nat)


@jax.jit
def kernel(h, edge_index, theta_w, theta_b, phi_w, phi_b):
    n_samples, n_points, n_dims = h.shape
    x = h.reshape(-1, n_dims)
    a, b = _node_matmuls(
        x,
        theta_w,
        phi_w,
        theta_b.reshape(1, F),
        phi_b.reshape(1, F),
    )
    c_part = _sc_scatter_max(b, _pack_edges(edge_index))
    c_bf = lax.bitcast_convert_type(c_part, jnp.bfloat16)
    c_nat = jnp.concatenate([c_bf[..., 0], c_bf[..., 1]], axis=-1)
    out = _combine(a, c_nat)
    return out.reshape(n_samples, n_points, F)


# speedup vs baseline: 2.6692x; 1.0888x over previous
import functools

import jax
import jax.numpy as jnp
from jax import lax
from jax.experimental import pallas as pl
from jax.experimental.pallas import tpu as pltpu
from jax.experimental.pallas import tpu_sc as plsc

N_NODES = 10000
N_EDGES = 320000
F = 128
L = 16

NC = 2
NS = 16
NPT = 632
NPAD = NS * NPT
E_PER_SC = N_EDGES // NC
CE = 8000
NCHUNK = E_PER_SC // CE
MB = 8192
GB = 128
NEG_INF_PAIR = -8323200


_ROWS_BLK = 2000


def _mm_kernel(x_ref, tw_ref, pw_ref, tb_ref, pb_ref, a_ref, b_ref):
    x = x_ref[...]
    tw = tw_ref[...]
    dn = (((1,), (1,)), ((), ()))
    a_ref[...] = (
        lax.dot_general(x, tw, dn, preferred_element_type=jnp.float32)
        + tb_ref[...] + pb_ref[...]
    )
    w = pw_ref[...] - tw
    b0 = lax.dot_general(x, w[:64], dn, preferred_element_type=jnp.float32)
    b1 = lax.dot_general(x, w[64:], dn, preferred_element_type=jnp.float32)
    u0 = lax.bitcast_convert_type(
        b0.astype(jnp.bfloat16), jnp.uint16
    ).astype(jnp.int32)
    u1 = lax.bitcast_convert_type(
        b1.astype(jnp.bfloat16), jnp.uint16
    ).astype(jnp.int32)
    b_ref[...] = u0 | (u1 << jnp.int32(16))


def _node_matmuls(x, theta_w, phi_w, theta_b, phi_b):
    n = x.shape[0]
    grid = n // _ROWS_BLK
    return pl.pallas_call(
        _mm_kernel,
        grid=(grid,),
        in_specs=[
            pl.BlockSpec((_ROWS_BLK, F), lambda i: (i, 0)),
            pl.BlockSpec((F, F), lambda i: (0, 0)),
            pl.BlockSpec((F, F), lambda i: (0, 0)),
            pl.BlockSpec((1, F), lambda i: (0, 0)),
            pl.BlockSpec((1, F), lambda i: (0, 0)),
        ],
        out_specs=[
            pl.BlockSpec((_ROWS_BLK, F), lambda i: (i, 0)),
            pl.BlockSpec((_ROWS_BLK, F // 2), lambda i: (i, 0)),
        ],
        out_shape=[
            jax.ShapeDtypeStruct((n, F), jnp.float32),
            jax.ShapeDtypeStruct((n, F // 2), jnp.int32),
        ],
    )(x, theta_w, phi_w, theta_b, phi_b)


_EROWS = N_EDGES // F


def _pack_kernel(e_ref, p_ref):
    e = e_ref[...]
    p_ref[...] = (e[1] << jnp.int32(16)) | e[0]


def _pack_edges(edge_index):
    e3 = edge_index.reshape(2, _EROWS, F)
    packed = pl.pallas_call(
        _pack_kernel,
        out_shape=jax.ShapeDtypeStruct((_EROWS, F), jnp.int32),
    )(e3)
    return packed.reshape(N_EDGES)


def _sc_body(b_hbm, edge_hbm, out_hbm, c_loc, shared_b,
             ech0, ech1, src_m, dst_m,
             rows0, rows1, sem_e0, sem_e1, sem_g0, sem_g1):
    c = lax.axis_index("c")
    s = lax.axis_index("s")
    lo = s * NPT

    rpt = N_NODES // NS
    pltpu.sync_copy(
        b_hbm.at[pl.ds(s * rpt, rpt)], shared_b.at[pl.ds(s * rpt, rpt)]
    )
    plsc.subcore_barrier()

    ech = (ech0, ech1)
    sem_e = (sem_e0, sem_e1)
    rows = (rows0, rows1)
    sem_g = (sem_g0, sem_g1)

    neg_inf = jnp.full((L,), NEG_INF_PAIR, dtype=jnp.int32)

    def init_row(r, carry):
        for j in range(F // (2 * L)):
            c_loc[r, pl.ds(j * L, L)] = neg_inf
        return carry

    lax.fori_loop(0, NPT + 1, init_row, 0)

    zero_v = jnp.zeros((L,), dtype=jnp.int32)

    def init_idx(g, carry):
        src_m[pl.ds(g * L, L)] = zero_v
        return carry

    lax.fori_loop(0, MB // L, init_idx, 0)

    lane = lax.iota(jnp.int32, L)
    sentv = jnp.full((L,), NPT, dtype=jnp.int32)
    ebase = c * E_PER_SC

    def fire_chunk(k, buf):
        off = ebase + k * CE
        pltpu.async_copy(edge_hbm.at[pl.ds(off, CE)], ech[buf], sem_e[buf])

    def wait_chunk(k, buf):
        off = ebase + k * CE
        pltpu.make_async_copy(
            edge_hbm.at[pl.ds(off, CE)], ech[buf], sem_e[buf]
        ).wait()

    def fire_gather(b, rbuf):
        pltpu.async_copy(
            shared_b.at[src_m.at[pl.ds(b * GB, GB)]], rows[rbuf], sem_g[rbuf]
        )

    def wait_gather(b, rbuf):
        pltpu.make_async_copy(
            shared_b.at[src_m.at[pl.ds(b * GB, GB)]], rows[rbuf], sem_g[rbuf]
        ).wait()

    def process_batch(b, rbuf):
        rbuf_ref = rows[rbuf]

        def group_body(g, carry2):
            e0 = b * GB + g * L
            dv = dst_m[pl.ds(e0, L)]
            for i in range(L):
                bc = jnp.take_along_axis(
                    dv,
                    jnp.full((L,), i, dtype=jnp.int32),
                    axis=0,
                    mode="promise_in_bounds",
                )
                for j in range(F // (2 * L)):
                    u = rbuf_ref[g * L + i, pl.ds(j * L, L)]
                    rv = plsc.bitcast(u, jnp.bfloat16)
                    colv = lane + (j * L)
                    cur_u = plsc.load_gather(c_loc, [bc, colv])
                    cur = plsc.bitcast(cur_u, jnp.bfloat16)
                    mx = jnp.maximum(cur, rv)
                    plsc.store_scatter(
                        c_loc, [bc, colv], plsc.bitcast(mx, jnp.int32)
                    )
            return carry2

        lax.fori_loop(0, GB // L, group_body, 0)

    def do_chunk(k, buf):
        wait_chunk(k, buf)

        def scan_pair(gp, cnt_v):
            for h in range(2):
                e0 = gp * (2 * L) + h * L
                u = ech[buf][pl.ds(e0, L)]
                dv = lax.shift_right_logical(u, jnp.int32(16))
                sv = u & jnp.int32(0xFFFF)
                dl = dv - lo
                msk = dl.astype(jnp.uint32) < jnp.uint32(NPT)
                inc = plsc.cumsum(msk.astype(jnp.int32))
                pos = cnt_v + inc - 1
                plsc.store_scatter(src_m, [pos], sv, mask=msk)
                plsc.store_scatter(dst_m, [pos], dl, mask=msk)
                cnt_v = cnt_v + plsc.all_reduce_population_count(msk)
            return cnt_v

        cnt_v = lax.fori_loop(
            0, CE // (2 * L), scan_pair, jnp.zeros((L,), jnp.int32)
        )
        cnt = jnp.max(cnt_v)

        for t in range(GB // L):
            plsc.store_scatter(dst_m, [cnt + lane + t * L], sentv)

        nb = (cnt + GB - 1) // GB

        @pl.when(nb > 0)
        def _():
            fire_gather(0, 0)

            def pair_body(p, carry):
                b0 = 2 * p

                @pl.when(b0 + 1 < nb)
                def _():
                    fire_gather(b0 + 1, 1)

                wait_gather(b0, 0)
                process_batch(b0, 0)

                @pl.when(b0 + 1 < nb)
                def _():
                    @pl.when(b0 + 2 < nb)
                    def _():
                        fire_gather(b0 + 2, 0)

                    wait_gather(b0 + 1, 1)
                    process_batch(b0 + 1, 1)

                return carry

            lax.fori_loop(0, (nb + 1) // 2, pair_body, 0)

    fire_chunk(0, 0)

    def pair_chunks(p, carry):
        a = 2 * p
        fire_chunk(a + 1, 1)
        do_chunk(a, 0)

        @pl.when(a + 2 < NCHUNK)
        def _():
            fire_chunk(a + 2, 0)

        do_chunk(a + 1, 1)
        return carry

    lax.fori_loop(0, NCHUNK // 2, pair_chunks, 0)

    pltpu.sync_copy(
        c_loc.at[pl.ds(0, NPT)], out_hbm.at[c, pl.ds(lo, NPT)]
    )


_sc_scatter_max = functools.partial(
    pl.kernel,
    out_type=jax.ShapeDtypeStruct((NC, NPAD, F // 2), jnp.int32),
    mesh=plsc.VectorSubcoreMesh(core_axis_name="c", subcore_axis_name="s"),
    compiler_params=pltpu.CompilerParams(
        needs_layout_passes=False, use_tc_tiling_on_sc=False
    ),
    scratch_types=[
        pltpu.VMEM((NPT + 1, F // 2), jnp.int32),
        pltpu.VMEM_SHARED((N_NODES, F // 2), jnp.int32),
        pltpu.VMEM((CE,), jnp.int32),
        pltpu.VMEM((CE,), jnp.int32),
        pltpu.VMEM((MB,), jnp.int32),
        pltpu.VMEM((MB,), jnp.int32),
        pltpu.VMEM((GB, F // 2), jnp.int32),
        pltpu.VMEM((GB, F // 2), jnp.int32),
        pltpu.SemaphoreType.DMA,
        pltpu.SemaphoreType.DMA,
        pltpu.SemaphoreType.DMA,
        pltpu.SemaphoreType.DMA,
    ],
)(_sc_body)


def _combine_kernel(a_ref, c_ref, o_ref):
    cm = jnp.maximum(c_ref[0], c_ref[1]).astype(jnp.float32)
    o_ref[...] = jnp.where(jnp.isfinite(cm), a_ref[...] + cm, 0.0)


def _combine(a, c_nat):
    n = a.shape[0]
    grid = n // _ROWS_BLK
    return pl.pallas_call(
        _combine_kernel,
        grid=(grid,),
        in_specs=[
            pl.BlockSpec((_ROWS_BLK, F), lambda i: (i, 0)),
            pl.BlockSpec((NC, _ROWS_BLK, F), lambda i: (0, i, 0)),
        ],
        out_specs=pl.BlockSpec((_ROWS_BLK, F), lambda i: (i, 0)),
        out_shape=jax.ShapeDtypeStruct((n, F), jnp.float32),
    )(a, c_nat)


@jax.jit
def kernel(h, edge_index, theta_w, theta_b, phi_w, phi_b):
    n_samples, n_points, n_dims = h.shape
    x = h.reshape(-1, n_dims)
    a, b = _node_matmuls(
        x,
        theta_w,
        phi_w,
        theta_b.reshape(1, F),
        phi_b.reshape(1, F),
    )
    c_part = _sc_scatter_max(b, _pack_edges(edge_index))
    c_bf = lax.bitcast_convert_type(c_part, jnp.bfloat16)
    c_nat = jnp.concatenate([c_bf[..., 0], c_bf[..., 1]], axis=-1)
    out = _combine(a, c_nat)
    return out.reshape(n_samples, n_points, F)
